# baseline (device time: 40947 ns/iter reference)
import jax
import jax.numpy as jnp
from jax import lax
from jax.experimental import pallas as pl
from jax.experimental.pallas import tpu as pltpu

N_DEV = 4
HEADS_PER_SHARD = 8
SQ = 256
SKV = 4096
DH = 128
DMODEL = 1024
BLOCK = 64
CHUNK = SQ // N_DEV
KV_TILE = 1024
N_TILES = SKV // KV_TILE
SCALE = 0.08838834764831843


def kernel(x, Wq, K_ext, V_ext, Wo):
    def body(x_ref, wq_ref, k_hbm, v_hbm, wo_ref, out_ref,
             k_buf, v_buf, kv_sems, ctx_ref, part_ref, part16_ref, rs_buf,
             red16_ref, ag_buf, rs_send, rs_recv, ag_send, ag_recv):
        my_pos = lax.axis_index("i")
        g0 = my_pos * HEADS_PER_SHARD

        def start_tile(t, slot):
            kc = pltpu.make_async_copy(
                k_hbm.at[0, pl.ds(t * KV_TILE, KV_TILE),
                         pl.ds(g0, HEADS_PER_SHARD), :],
                k_buf.at[slot], kv_sems.at[slot, 0])
            vc = pltpu.make_async_copy(
                v_hbm.at[0, pl.ds(t * KV_TILE, KV_TILE),
                         pl.ds(g0, HEADS_PER_SHARD), :],
                v_buf.at[slot], kv_sems.at[slot, 1])
            kc.start()
            vc.start()
            return kc, vc

        pending = [start_tile(0, 0), start_tile(1, 1)]

        q = jnp.dot(
            x_ref[0].astype(jnp.bfloat16), wq_ref[...].astype(jnp.bfloat16),
            preferred_element_type=jnp.float32)
        q = (q * SCALE).astype(jnp.bfloat16)

        qb = lax.broadcasted_iota(jnp.int32, (SQ, SKV), 0) // BLOCK
        kb = lax.broadcasted_iota(jnp.int32, (SQ, SKV), 1) // BLOCK
        mask = (qb == kb) | (kb == 0) | (((qb + kb) % 3) == 0)
        neg = jnp.where(mask, 0.0, -1e9).astype(jnp.float32)

        ctx_acc = [jnp.zeros((SQ, DH), jnp.float32)
                   for _ in range(HEADS_PER_SHARD)]
        den_acc = [jnp.zeros((SQ, 1), jnp.float32)
                   for _ in range(HEADS_PER_SHARD)]
        for t in range(N_TILES):
            slot = t % 2
            pending[slot][0].wait()
            pending[slot][1].wait()
            kt = jnp.swapaxes(
                k_buf[slot].astype(jnp.bfloat16), 0, 1)
            vt = jnp.swapaxes(v_buf[slot].astype(jnp.bfloat16), 0, 1)
            neg_t = neg[:, t * KV_TILE:(t + 1) * KV_TILE]
            for h in range(HEADS_PER_SHARD):
                qh = q[:, h * DH:(h + 1) * DH]
                scores = lax.dot_general(
                    qh, kt[h],
                    (((1,), (1,)), ((), ())),
                    preferred_element_type=jnp.float32,
                ) + neg_t
                w = jnp.exp(scores)
                den_acc[h] = den_acc[h] + jnp.sum(w, axis=-1, keepdims=True)
                ctx_acc[h] = ctx_acc[h] + jnp.dot(
                    w.astype(jnp.bfloat16), vt[h],
                    preferred_element_type=jnp.float32)
            if t + 2 < N_TILES:
                pending[slot] = start_tile(t + 2, slot)
        for h in range(HEADS_PER_SHARD):
            ctx_ref[:, h * DH:(h + 1) * DH] = (
                ctx_acc[h] / den_acc[h]).astype(jnp.bfloat16)

        barrier_sem = pltpu.get_barrier_semaphore()
        for d in range(1, N_DEV):
            pl.semaphore_signal(
                barrier_sem, inc=1,
                device_id=((my_pos + d) % N_DEV,),
                device_id_type=pl.DeviceIdType.MESH,
            )
        pl.semaphore_wait(barrier_sem, N_DEV - 1)

        wo16 = wo_ref[...].astype(jnp.bfloat16)
        rs_ops = []
        for d in (2, 1, 3):
            t = (my_pos + d) % N_DEV
            pchunk = jnp.dot(
                ctx_ref[pl.ds(t * CHUNK, CHUNK), :], wo16,
                preferred_element_type=jnp.float32)
            part16_ref[pl.ds(t * CHUNK, CHUNK), :] = (
                pchunk.astype(jnp.bfloat16))
            op = pltpu.make_async_remote_copy(
                src_ref=part16_ref.at[pl.ds(t * CHUNK, CHUNK), :],
                dst_ref=rs_buf.at[my_pos],
                send_sem=rs_send.at[d - 1],
                recv_sem=rs_recv.at[my_pos],
                device_id=(t,),
                device_id_type=pl.DeviceIdType.MESH,
            )
            op.start()
            rs_ops.append(op)

        red = jnp.dot(
            ctx_ref[pl.ds(my_pos * CHUNK, CHUNK), :], wo16,
            preferred_element_type=jnp.float32)
        for d in range(1, N_DEV):
            s = (my_pos + d) % N_DEV
            recv = pltpu.make_async_remote_copy(
                src_ref=part16_ref.at[pl.ds(0, CHUNK), :],
                dst_ref=rs_buf.at[s],
                send_sem=rs_send.at[0],
                recv_sem=rs_recv.at[s],
                device_id=(s,),
                device_id_type=pl.DeviceIdType.MESH,
            )
            recv.wait_recv()
            red = red + rs_buf[s].astype(jnp.float32)

        red16_ref[...] = red.astype(jnp.bfloat16)
        ag_ops = []
        for d in (2, 1, 3):
            t = (my_pos + d) % N_DEV
            op = pltpu.make_async_remote_copy(
                src_ref=red16_ref,
                dst_ref=ag_buf.at[my_pos],
                send_sem=ag_send.at[d - 1],
                recv_sem=ag_recv.at[my_pos],
                device_id=(t,),
                device_id_type=pl.DeviceIdType.MESH,
            )
            op.start()
            ag_ops.append(op)

        out_ref[0, pl.ds(my_pos * CHUNK, CHUNK), :] = red

        for d in range(1, N_DEV):
            s = (my_pos + d) % N_DEV
            recv = pltpu.make_async_remote_copy(
                src_ref=red16_ref,
                dst_ref=ag_buf.at[s],
                send_sem=ag_send.at[0],
                recv_sem=ag_recv.at[s],
                device_id=(s,),
                device_id_type=pl.DeviceIdType.MESH,
            )
            recv.wait_recv()
            out_ref[0, pl.ds(s * CHUNK, CHUNK), :] = (
                ag_buf[s].astype(jnp.float32))

        for op in rs_ops + ag_ops:
            op.wait_send()

    return pl.pallas_call(
        body,
        out_shape=jax.ShapeDtypeStruct((1, SQ, DMODEL), jnp.float32),
        in_specs=[
            pl.BlockSpec(memory_space=pltpu.VMEM),
            pl.BlockSpec(memory_space=pltpu.VMEM),
            pl.BlockSpec(memory_space=pl.ANY),
            pl.BlockSpec(memory_space=pl.ANY),
            pl.BlockSpec(memory_space=pltpu.VMEM),
        ],
        out_specs=pl.BlockSpec(memory_space=pltpu.VMEM),
        scratch_shapes=[
            pltpu.VMEM((2, KV_TILE, HEADS_PER_SHARD, DH), jnp.float32),
            pltpu.VMEM((2, KV_TILE, HEADS_PER_SHARD, DH), jnp.float32),
            pltpu.SemaphoreType.DMA((2, 2)),
            pltpu.VMEM((SQ, DMODEL), jnp.bfloat16),
            pltpu.VMEM((SQ, DMODEL), jnp.float32),
            pltpu.VMEM((SQ, DMODEL), jnp.bfloat16),
            pltpu.VMEM((N_DEV, CHUNK, DMODEL), jnp.bfloat16),
            pltpu.VMEM((CHUNK, DMODEL), jnp.bfloat16),
            pltpu.VMEM((N_DEV, CHUNK, DMODEL), jnp.bfloat16),
            pltpu.SemaphoreType.DMA((N_DEV - 1,)),
            pltpu.SemaphoreType.DMA((N_DEV,)),
            pltpu.SemaphoreType.DMA((N_DEV - 1,)),
            pltpu.SemaphoreType.DMA((N_DEV,)),
        ],
        compiler_params=pltpu.CompilerParams(
            collective_id=0, vmem_limit_bytes=100 * 1024 * 1024),
    )(x, Wq, K_ext, V_ext, Wo)
